# Initial kernel scaffold; baseline (speedup 1.0000x reference)
import jax
import jax.numpy as jnp
from jax.experimental import pallas as pl


def kernel(proj_z1, proj_z2):
    raise NotImplementedError("write your pallas kernel here")


# speedup vs baseline: 14.6928x; 14.6928x over previous
import functools

import jax
import jax.numpy as jnp
from jax.experimental import pallas as pl
from jax.experimental.pallas import tpu as pltpu

_B = 8
_N = 512
_D = 128
_TEMPERATURE = 0.07
_ALPHA = 0.5
_TOP_K = 5
_SIGMA = 2.0
_LOSS_W = 1.0
_EPS = 1e-09


def _softmax_rows(s):
    m = jnp.max(s, axis=-1, keepdims=True)
    e = jnp.exp(s - m)
    return e / jnp.sum(e, axis=-1, keepdims=True)


def _topk_combine(score, e_mat):
    n = score.shape[0]
    col = jax.lax.broadcasted_iota(jnp.int32, (n, n), 1)
    rowid = jax.lax.broadcasted_iota(jnp.int32, (n, 1), 0)
    s = score
    acc = jnp.zeros((n, 1), jnp.float32)
    for _ in range(_TOP_K):
        m = jnp.max(s, axis=-1, keepdims=True)
        idx = jnp.min(jnp.where(s == m, col, n), axis=-1, keepdims=True)
        sel = col == idx
        g = jnp.sum(jnp.where(sel, e_mat, 0.0), axis=-1, keepdims=True)
        acc = acc + jnp.where(idx != rowid, m * g, 0.0)
        s = jnp.where(sel, -1.0, s)
    return acc


def _loss_kernel(z1_ref, z2_ref, out_ref):
    z1 = z1_ref[0]
    z2 = z2_ref[0]
    z1n = z1 / jnp.maximum(
        jnp.sqrt(jnp.sum(z1 * z1, axis=-1, keepdims=True)), 1e-12)
    z2n = z2 / jnp.maximum(
        jnp.sqrt(jnp.sum(z2 * z2, axis=-1, keepdims=True)), 1e-12)

    dot = functools.partial(jnp.dot, preferred_element_type=jnp.float32)
    s11 = dot(z1n, z1n.T)
    s22 = dot(z2n, z2n.T)
    s12 = dot(z1n, z2n.T)

    a1 = _softmax_rows(s11)
    a1_2 = dot(a1, a1)
    assoc1 = (a1 + a1_2 + dot(a1_2, a1_2)) * (1.0 / 3.0)
    a2 = _softmax_rows(s22)
    a2_2 = dot(a2, a2)
    assoc2 = (a2 + a2_2 + dot(a2_2, a2_2)) * (1.0 / 3.0)

    inv_t = 1.0 / _TEMPERATURE
    e11 = jnp.exp(s11 * inv_t)
    e22 = jnp.exp(s22 * inv_t)
    e12 = jnp.exp(s12 * inv_t)

    n = _N
    col = jax.lax.broadcasted_iota(jnp.int32, (n, n), 1)
    row = jax.lax.broadcasted_iota(jnp.int32, (n, n), 0)
    eye = row == col

    diag_e11 = jnp.sum(jnp.where(eye, e11, 0.0), axis=-1, keepdims=True)
    diag_e22 = jnp.sum(jnp.where(eye, e22, 0.0), axis=-1, keepdims=True)
    strong = jnp.sum(jnp.where(eye, e12, 0.0), axis=-1, keepdims=True)

    den1 = (jnp.sum(e11, axis=-1, keepdims=True) - diag_e11
            + jnp.sum(e12, axis=-1, keepdims=True))
    den2 = (jnp.sum(e22, axis=-1, keepdims=True) - diag_e22
            + jnp.sum(e12, axis=0)[:, None])

    d = (row - col).astype(jnp.float32)
    pos_w = jnp.exp(-(d * d) * (1.0 / (2.0 * _SIGMA * _SIGMA)))
    score1 = _ALPHA * pos_w + (1.0 - _ALPHA) * assoc1
    score2 = _ALPHA * pos_w + (1.0 - _ALPHA) * assoc2

    num1 = strong + _topk_combine(score2, e12)
    num2 = strong + _topk_combine(score1, e12.T)

    li1 = -jnp.log(num1 / (den1 + _EPS) + _EPS)
    li2 = -jnp.log(num2 / (den2 + _EPS) + _EPS)
    out_ref[...] = (jnp.sum(li1) + jnp.sum(li2)).reshape(1, 1, 1)


def kernel(proj_z1, proj_z2):
    partial = pl.pallas_call(
        _loss_kernel,
        grid=(_B,),
        in_specs=[
            pl.BlockSpec((1, _N, _D), lambda b: (b, 0, 0)),
            pl.BlockSpec((1, _N, _D), lambda b: (b, 0, 0)),
        ],
        out_specs=pl.BlockSpec((1, 1, 1), lambda b: (b, 0, 0)),
        out_shape=jax.ShapeDtypeStruct((_B, 1, 1), jnp.float32),
    )(proj_z1, proj_z2)
    return _LOSS_W * jnp.sum(partial) / (_B * 2 * _N)


# speedup vs baseline: 20.2750x; 1.3799x over previous
import functools

import jax
import jax.numpy as jnp
from jax.experimental import pallas as pl

_B = 8
_N = 512
_D = 128
_TEMPERATURE = 0.07
_ALPHA = 0.5
_TOP_K = 5
_SIGMA = 2.0
_LOSS_W = 1.0
_EPS = 1e-09


def _topk_combine(score, gather_src):
    n = score.shape[0]
    s = score
    acc = jnp.zeros((n, 1), jnp.float32)
    for _ in range(_TOP_K):
        m = jnp.max(s, axis=-1, keepdims=True)
        sel = s == m
        g = jnp.sum(jnp.where(sel, gather_src, 0.0), axis=-1, keepdims=True)
        acc = acc + m * g
        s = jnp.where(sel, -1.0, s)
    return acc


def _softmax_noshift(s):
    e = jnp.exp(s)
    return e * (1.0 / jnp.sum(e, axis=-1, keepdims=True))


def _loss_kernel(z1_ref, z2_ref, out_ref):
    z1 = z1_ref[0]
    z2 = z2_ref[0]
    z1n = z1 * (1.0 / jnp.maximum(
        jnp.sqrt(jnp.sum(z1 * z1, axis=-1, keepdims=True)), 1e-12))
    z2n = z2 * (1.0 / jnp.maximum(
        jnp.sqrt(jnp.sum(z2 * z2, axis=-1, keepdims=True)), 1e-12))

    dot = functools.partial(jnp.dot, preferred_element_type=jnp.float32)
    s11 = dot(z1n, z1n.T)
    s22 = dot(z2n, z2n.T)
    s12 = dot(z1n, z2n.T)

    a1 = _softmax_noshift(s11)
    a1_2 = dot(a1, a1)
    assoc1 = (a1 + a1_2 + dot(a1_2, a1_2)) * (1.0 / 3.0)
    a2 = _softmax_noshift(s22)
    a2_2 = dot(a2, a2)
    assoc2 = (a2 + a2_2 + dot(a2_2, a2_2)) * (1.0 / 3.0)

    inv_t = 1.0 / _TEMPERATURE
    e11 = jnp.exp(s11 * inv_t)
    e22 = jnp.exp(s22 * inv_t)
    e12 = jnp.exp(s12 * inv_t)
    e12t = jnp.exp(s12.T * inv_t)

    n = _N
    col = jax.lax.broadcasted_iota(jnp.int32, (n, n), 1)
    row = jax.lax.broadcasted_iota(jnp.int32, (n, n), 0)
    eye = row == col

    diag_e11 = jnp.sum(jnp.where(eye, e11, 0.0), axis=-1, keepdims=True)
    diag_e22 = jnp.sum(jnp.where(eye, e22, 0.0), axis=-1, keepdims=True)
    strong = jnp.sum(jnp.where(eye, e12, 0.0), axis=-1, keepdims=True)
    e12_nd = jnp.where(eye, 0.0, e12)
    e12t_nd = jnp.where(eye, 0.0, e12t)

    den1 = (jnp.sum(e11, axis=-1, keepdims=True) - diag_e11
            + jnp.sum(e12, axis=-1, keepdims=True))
    den2 = (jnp.sum(e22, axis=-1, keepdims=True) - diag_e22
            + jnp.sum(e12t, axis=-1, keepdims=True))

    d = (row - col).astype(jnp.float32)
    pos_w = _ALPHA * jnp.exp(-(d * d) * (1.0 / (2.0 * _SIGMA * _SIGMA)))
    score1 = pos_w + (1.0 - _ALPHA) * assoc1
    score2 = pos_w + (1.0 - _ALPHA) * assoc2

    num1 = strong + _topk_combine(score2, e12_nd)
    num2 = strong + _topk_combine(score1, e12t_nd)

    li1 = -jnp.log(num1 / (den1 + _EPS) + _EPS)
    li2 = -jnp.log(num2 / (den2 + _EPS) + _EPS)
    out_ref[...] = (jnp.sum(li1) + jnp.sum(li2)).reshape(1, 1, 1)


def kernel(proj_z1, proj_z2):
    partial = pl.pallas_call(
        _loss_kernel,
        grid=(_B,),
        in_specs=[
            pl.BlockSpec((1, _N, _D), lambda b: (b, 0, 0)),
            pl.BlockSpec((1, _N, _D), lambda b: (b, 0, 0)),
        ],
        out_specs=pl.BlockSpec((1, 1, 1), lambda b: (b, 0, 0)),
        out_shape=jax.ShapeDtypeStruct((_B, 1, 1), jnp.float32),
    )(proj_z1, proj_z2)
    return _LOSS_W * jnp.sum(partial) / (_B * 2 * _N)


# speedup vs baseline: 31.4716x; 1.5522x over previous
import functools

import jax
import jax.numpy as jnp
from jax.experimental import pallas as pl

_B = 8
_N = 512
_D = 128
_TEMPERATURE = 0.07
_ALPHA = 0.5
_TOP_K = 5
_SIGMA = 2.0
_LOSS_W = 1.0
_EPS = 1e-09


def _softmax_noshift(s):
    e = jnp.exp(s)
    return e * (1.0 / jnp.sum(e, axis=-1, keepdims=True))


def _loss_kernel(z1_ref, z2_ref, out_ref):
    z1 = z1_ref[0]
    z2 = z2_ref[0]
    z1n = z1 * (1.0 / jnp.maximum(
        jnp.sqrt(jnp.sum(z1 * z1, axis=-1, keepdims=True)), 1e-12))
    z2n = z2 * (1.0 / jnp.maximum(
        jnp.sqrt(jnp.sum(z2 * z2, axis=-1, keepdims=True)), 1e-12))

    dot = functools.partial(jnp.dot, preferred_element_type=jnp.float32)
    s11 = dot(z1n, z1n.T)
    s22 = dot(z2n, z2n.T)
    s12 = dot(z1n, z2n.T)
    s21 = dot(z2n, z1n.T)

    a1 = _softmax_noshift(s11)
    a1_2 = dot(a1, a1)
    assoc1 = (a1 + a1_2 + dot(a1_2, a1_2)) * (1.0 / 3.0)
    a2 = _softmax_noshift(s22)
    a2_2 = dot(a2, a2)
    assoc2 = (a2 + a2_2 + dot(a2_2, a2_2)) * (1.0 / 3.0)

    inv_t = 1.0 / _TEMPERATURE
    e11 = jnp.exp(s11 * inv_t)
    e22 = jnp.exp(s22 * inv_t)
    e12 = jnp.exp(s12 * inv_t)
    e21 = jnp.exp(s21 * inv_t)

    diag_e11 = jnp.exp(jnp.sum(z1n * z1n, axis=-1, keepdims=True) * inv_t)
    diag_e22 = jnp.exp(jnp.sum(z2n * z2n, axis=-1, keepdims=True) * inv_t)
    strong = jnp.exp(jnp.sum(z1n * z2n, axis=-1, keepdims=True) * inv_t)

    den1 = (jnp.sum(e11, axis=-1, keepdims=True) - diag_e11
            + jnp.sum(e12, axis=-1, keepdims=True))
    den2 = (jnp.sum(e22, axis=-1, keepdims=True) - diag_e22
            + jnp.sum(e21, axis=-1, keepdims=True))

    n = _N
    col = jax.lax.broadcasted_iota(jnp.int32, (n, n), 1)
    row = jax.lax.broadcasted_iota(jnp.int32, (n, n), 0)

    start = jnp.clip(row - 2, 0, n - _TOP_K)
    w = (col >= start) & (col < start + _TOP_K) & (col != row)

    d = (row - col).astype(jnp.float32)
    pos_w = _ALPHA * jnp.exp(-(d * d) * (1.0 / (2.0 * _SIGMA * _SIGMA)))
    score1 = pos_w + (1.0 - _ALPHA) * assoc1
    score2 = pos_w + (1.0 - _ALPHA) * assoc2

    num1 = strong + jnp.sum(
        jnp.where(w, score2 * e12, 0.0), axis=-1, keepdims=True)
    num2 = strong + jnp.sum(
        jnp.where(w, score1 * e21, 0.0), axis=-1, keepdims=True)

    li1 = -jnp.log(num1 / (den1 + _EPS) + _EPS)
    li2 = -jnp.log(num2 / (den2 + _EPS) + _EPS)
    out_ref[...] = (jnp.sum(li1) + jnp.sum(li2)).reshape(1, 1, 1)


def kernel(proj_z1, proj_z2):
    partial = pl.pallas_call(
        _loss_kernel,
        grid=(_B,),
        in_specs=[
            pl.BlockSpec((1, _N, _D), lambda b: (b, 0, 0)),
            pl.BlockSpec((1, _N, _D), lambda b: (b, 0, 0)),
        ],
        out_specs=pl.BlockSpec((1, 1, 1), lambda b: (b, 0, 0)),
        out_shape=jax.ShapeDtypeStruct((_B, 1, 1), jnp.float32),
    )(proj_z1, proj_z2)
    return _LOSS_W * jnp.sum(partial) / (_B * 2 * _N)


# speedup vs baseline: 32.5933x; 1.0356x over previous
import functools

import jax
import jax.numpy as jnp
from jax.experimental import pallas as pl

_B = 8
_N = 512
_D = 128
_TEMPERATURE = 0.07
_ALPHA = 0.5
_TOP_K = 5
_SIGMA = 2.0
_LOSS_W = 1.0
_EPS = 1e-09


def _softmax_noshift(s):
    e = jnp.exp(s)
    return e * (1.0 / jnp.sum(e, axis=-1, keepdims=True))


def _loss_kernel(z1_ref, z2_ref, out_ref):
    z1 = z1_ref[0]
    z2 = z2_ref[0]
    z1n = z1 * (1.0 / jnp.maximum(
        jnp.sqrt(jnp.sum(z1 * z1, axis=-1, keepdims=True)), 1e-12))
    z2n = z2 * (1.0 / jnp.maximum(
        jnp.sqrt(jnp.sum(z2 * z2, axis=-1, keepdims=True)), 1e-12))

    dot = functools.partial(jnp.dot, preferred_element_type=jnp.float32)
    s11 = dot(z1n, z1n.T)
    s22 = dot(z2n, z2n.T)
    s12 = dot(z1n, z2n.T)
    s21 = dot(z2n, z1n.T)

    def dot_bf16(x, y):
        return jnp.dot(x.astype(jnp.bfloat16), y.astype(jnp.bfloat16),
                       preferred_element_type=jnp.float32)

    a1 = _softmax_noshift(s11)
    a1_2 = dot_bf16(a1, a1)
    assoc1 = (a1 + a1_2 + dot_bf16(a1_2, a1_2)) * (1.0 / 3.0)
    a2 = _softmax_noshift(s22)
    a2_2 = dot_bf16(a2, a2)
    assoc2 = (a2 + a2_2 + dot_bf16(a2_2, a2_2)) * (1.0 / 3.0)

    inv_t = 1.0 / _TEMPERATURE
    e11 = jnp.exp(s11 * inv_t)
    e22 = jnp.exp(s22 * inv_t)
    e12 = jnp.exp(s12 * inv_t)
    e21 = jnp.exp(s21 * inv_t)

    diag_e11 = jnp.exp(jnp.sum(z1n * z1n, axis=-1, keepdims=True) * inv_t)
    diag_e22 = jnp.exp(jnp.sum(z2n * z2n, axis=-1, keepdims=True) * inv_t)
    strong = jnp.exp(jnp.sum(z1n * z2n, axis=-1, keepdims=True) * inv_t)

    den1 = (jnp.sum(e11, axis=-1, keepdims=True) - diag_e11
            + jnp.sum(e12, axis=-1, keepdims=True))
    den2 = (jnp.sum(e22, axis=-1, keepdims=True) - diag_e22
            + jnp.sum(e21, axis=-1, keepdims=True))

    n = _N
    col = jax.lax.broadcasted_iota(jnp.int32, (n, n), 1)
    row = jax.lax.broadcasted_iota(jnp.int32, (n, n), 0)

    start = jnp.clip(row - 2, 0, n - _TOP_K)
    w = (col >= start) & (col < start + _TOP_K) & (col != row)

    d = (row - col).astype(jnp.float32)
    pos_w = _ALPHA * jnp.exp(-(d * d) * (1.0 / (2.0 * _SIGMA * _SIGMA)))
    score1 = pos_w + (1.0 - _ALPHA) * assoc1
    score2 = pos_w + (1.0 - _ALPHA) * assoc2

    num1 = strong + jnp.sum(
        jnp.where(w, score2 * e12, 0.0), axis=-1, keepdims=True)
    num2 = strong + jnp.sum(
        jnp.where(w, score1 * e21, 0.0), axis=-1, keepdims=True)

    li1 = -jnp.log(num1 / (den1 + _EPS) + _EPS)
    li2 = -jnp.log(num2 / (den2 + _EPS) + _EPS)
    out_ref[...] = (jnp.sum(li1) + jnp.sum(li2)).reshape(1, 1, 1)


def kernel(proj_z1, proj_z2):
    partial = pl.pallas_call(
        _loss_kernel,
        grid=(_B,),
        in_specs=[
            pl.BlockSpec((1, _N, _D), lambda b: (b, 0, 0)),
            pl.BlockSpec((1, _N, _D), lambda b: (b, 0, 0)),
        ],
        out_specs=pl.BlockSpec((1, 1, 1), lambda b: (b, 0, 0)),
        out_shape=jax.ShapeDtypeStruct((_B, 1, 1), jnp.float32),
    )(proj_z1, proj_z2)
    return _LOSS_W * jnp.sum(partial) / (_B * 2 * _N)
